# Initial kernel scaffold; baseline (speedup 1.0000x reference)
import jax
import jax.numpy as jnp
from jax.experimental import pallas as pl


def kernel(context, A_tables, C_last):
    raise NotImplementedError("write your pallas kernel here")


# speedup vs baseline: 28.8691x; 28.8691x over previous
import functools

import jax
import jax.numpy as jnp
from jax import lax
from jax.experimental import pallas as pl
from jax.experimental.pallas import tpu as pltpu
from jax.experimental.pallas import tpu_sc as plsc


def _gather_sums(idx, t1, t2, t3, P, S, E):
    info = plsc.get_sparse_core_info()
    NW = info.num_cores * info.num_subcores
    NC = info.num_cores
    PW = P // NW
    CP = 64
    NCH = PW // CP
    RB = CP * S // 128
    assert P % NW == 0 and PW % CP == 0 and (CP * S) % 128 == 0

    mesh = plsc.VectorSubcoreMesh(core_axis_name="c", subcore_axis_name="s")
    out_t = jax.ShapeDtypeStruct((P, E), jnp.float32)

    @functools.partial(
        pl.kernel,
        mesh=mesh,
        out_type=(out_t, out_t, out_t),
        compiler_params=pltpu.CompilerParams(use_tc_tiling_on_sc=False),
        scratch_types=[
            pltpu.VMEM((CP * S,), jnp.int32),
            pltpu.VMEM((CP * S, E), jnp.float32),
            pltpu.VMEM((CP, E), jnp.float32),
            pltpu.SemaphoreType.DMA,
        ],
    )
    def gsum(idx_hbm, t1_hbm, t2_hbm, t3_hbm, o1_hbm, o2_hbm, o3_hbm,
             idx_v, rows_v, acc_v, sem):
        wid = lax.axis_index("s") * NC + lax.axis_index("c")

        def chunk_body(c, carry):
            pair0 = wid * PW + c * CP
            pltpu.sync_copy(idx_hbm.at[pl.ds(pair0 * S, CP * S)], idx_v)
            for t_hbm, o_hbm in ((t1_hbm, o1_hbm), (t2_hbm, o2_hbm),
                                 (t3_hbm, o3_hbm)):
                copies = [
                    pltpu.async_copy(t_hbm.at[idx_v.at[pl.ds(j * 128, 128)]],
                                     rows_v.at[pl.ds(j * 128, 128)], sem)
                    for j in range(RB)
                ]
                for cp in copies:
                    cp.wait()

                def pair_body(p, carry2):
                    r0 = p * S
                    lo = rows_v[r0, pl.ds(0, 16)]
                    hi = rows_v[r0, pl.ds(16, 16)]
                    for s in range(1, S):
                        lo = lo + rows_v[r0 + s, pl.ds(0, 16)]
                        hi = hi + rows_v[r0 + s, pl.ds(16, 16)]
                    acc_v[p, pl.ds(0, 16)] = lo
                    acc_v[p, pl.ds(16, 16)] = hi
                    return carry2

                lax.fori_loop(0, CP, pair_body, 0, unroll=False)
                pltpu.sync_copy(acc_v, o_hbm.at[pl.ds(pair0, CP)])
            return carry

        lax.fori_loop(0, NCH, chunk_body, 0, unroll=False)

    return gsum(idx, t1, t2, t3)


def _epilogue(g1, g2, gc, E):
    P = g1.shape[0]
    R = 512
    inv_e = 1.0 / E

    def body(g1_ref, g2_ref, gc_ref, o_ref):
        a = g1_ref[...]
        b = g2_ref[...]
        c = gc_ref[...]
        q1 = a * inv_e
        t1 = a * q1
        t1 = t1 - jnp.max(t1, axis=-1, keepdims=True)
        e1 = jnp.exp(t1)
        a1 = e1 / jnp.sum(e1, axis=-1, keepdims=True)
        q2 = q1 + b * a1
        t2 = b * q2
        t2 = t2 - jnp.max(t2, axis=-1, keepdims=True)
        e2 = jnp.exp(t2)
        o_ref[...] = c * (e2 / jnp.sum(e2, axis=-1, keepdims=True))

    spec = pl.BlockSpec((R, E), lambda i: (i, 0))
    return pl.pallas_call(
        body,
        grid=(P // R,),
        in_specs=[spec, spec, spec],
        out_specs=spec,
        out_shape=jax.ShapeDtypeStruct((P, E), jnp.float32),
    )(g1, g2, gc)


def kernel(context, A_tables, C_last):
    B, M, S = context.shape
    hops, V, E = A_tables.shape
    P = B * M
    assert hops == 3 and E == 32 and (P * S) % 128 == 0

    idx = context.reshape(P * S)
    g1, g2, gc = _gather_sums(idx, A_tables[1], A_tables[2], C_last,
                              P, S, E)
    return _epilogue(g1, g2, gc, E).reshape(B, M, E)
